# Initial kernel scaffold; baseline (speedup 1.0000x reference)
import jax
import jax.numpy as jnp
from jax.experimental import pallas as pl


def kernel(x, adj, W0, W1, W2):
    raise NotImplementedError("write your pallas kernel here")


# speedup vs baseline: 1.0363x; 1.0363x over previous
import jax
import jax.numpy as jnp
from jax.experimental import pallas as pl


_BM = 256


def _layer_body(adj_ref, h_ref, w_ref, out_ref):
    t = jnp.dot(adj_ref[...], h_ref[...], preferred_element_type=jnp.float32)
    t = jnp.dot(t, w_ref[...], preferred_element_type=jnp.float32)
    out_ref[...] = jnp.maximum(t, 0.0).astype(out_ref.dtype)


def _layer(adj, h, w, out_dtype):
    m, k = adj.shape
    d = w.shape[1]
    return pl.pallas_call(
        _layer_body,
        grid=(pl.cdiv(m, _BM),),
        in_specs=[
            pl.BlockSpec((_BM, k), lambda i: (i, 0)),
            pl.BlockSpec((k, d), lambda i: (0, 0)),
            pl.BlockSpec((d, d), lambda i: (0, 0)),
        ],
        out_specs=pl.BlockSpec((_BM, d), lambda i: (i, 0)),
        out_shape=jax.ShapeDtypeStruct((m, d), out_dtype),
    )(adj, h, w)


def kernel(x, adj, W0, W1, W2):
    h = _layer(adj, x, W0, jnp.float32)
    h = _layer(adj, h, W1, jnp.float32)
    return _layer(adj, h, W2, jnp.float32)


# speedup vs baseline: 1.0502x; 1.0135x over previous
import jax
import jax.numpy as jnp
from jax.experimental import pallas as pl


_BM = 256


def _layer1_body(adj_ref, h_ref, w_ref, out_ref, adj16_ref):
    a16 = adj_ref[...].astype(jnp.bfloat16)
    adj16_ref[...] = a16
    t = jnp.dot(a16, h_ref[...], preferred_element_type=jnp.float32)
    t = jnp.dot(t, w_ref[...], preferred_element_type=jnp.float32)
    out_ref[...] = jnp.maximum(t, 0.0).astype(out_ref.dtype)


def _layer1(adj, h, w):
    m, k = adj.shape
    d = w.shape[1]
    return pl.pallas_call(
        _layer1_body,
        grid=(pl.cdiv(m, _BM),),
        in_specs=[
            pl.BlockSpec((_BM, k), lambda i: (i, 0)),
            pl.BlockSpec((k, d), lambda i: (0, 0)),
            pl.BlockSpec((d, d), lambda i: (0, 0)),
        ],
        out_specs=[
            pl.BlockSpec((_BM, d), lambda i: (i, 0)),
            pl.BlockSpec((_BM, k), lambda i: (i, 0)),
        ],
        out_shape=[
            jax.ShapeDtypeStruct((m, d), jnp.bfloat16),
            jax.ShapeDtypeStruct((m, k), jnp.bfloat16),
        ],
    )(adj, h, w)


def _layer_body(adj_ref, h_ref, w_ref, out_ref):
    t = jnp.dot(adj_ref[...], h_ref[...], preferred_element_type=jnp.float32)
    t = jnp.dot(t, w_ref[...], preferred_element_type=jnp.float32)
    out_ref[...] = jnp.maximum(t, 0.0).astype(out_ref.dtype)


def _layer(adj, h, w, out_dtype):
    m, k = adj.shape
    d = w.shape[1]
    return pl.pallas_call(
        _layer_body,
        grid=(pl.cdiv(m, _BM),),
        in_specs=[
            pl.BlockSpec((_BM, k), lambda i: (i, 0)),
            pl.BlockSpec((k, d), lambda i: (0, 0)),
            pl.BlockSpec((d, d), lambda i: (0, 0)),
        ],
        out_specs=pl.BlockSpec((_BM, d), lambda i: (i, 0)),
        out_shape=jax.ShapeDtypeStruct((m, d), out_dtype),
    )(adj, h, w)


def kernel(x, adj, W0, W1, W2):
    h, adj16 = _layer1(adj, x.astype(jnp.bfloat16), W0)
    h = _layer(adj16, h, W1, jnp.bfloat16)
    return _layer(adj16, h, W2, jnp.float32)


# speedup vs baseline: 1.1552x; 1.1000x over previous
import jax
import jax.numpy as jnp
from jax.experimental import pallas as pl


_BM = 256


def _layer1_body(adj_ref, h_ref, w_ref, out_ref, adj16_ref):
    a16 = adj_ref[...].astype(jnp.bfloat16)
    adj16_ref[...] = a16
    t = jnp.dot(a16, h_ref[...], preferred_element_type=jnp.float32)
    t = jnp.dot(t, w_ref[...], preferred_element_type=jnp.float32)
    out_ref[...] = jnp.maximum(t, 0.0).astype(out_ref.dtype)


def _layer1(adj, h, w):
    m, k = adj.shape
    d = w.shape[1]
    return pl.pallas_call(
        _layer1_body,
        grid=(pl.cdiv(m, _BM),),
        in_specs=[
            pl.BlockSpec((_BM, k), lambda i: (i, 0)),
            pl.BlockSpec((k, d), lambda i: (0, 0)),
            pl.BlockSpec((d, d), lambda i: (0, 0)),
        ],
        out_specs=[
            pl.BlockSpec((_BM, d), lambda i: (i, 0)),
            pl.BlockSpec((_BM, k), lambda i: (i, 0)),
        ],
        out_shape=[
            jax.ShapeDtypeStruct((m, d), jnp.bfloat16),
            jax.ShapeDtypeStruct((m, k), jnp.bfloat16),
        ],
    )(adj, h, w)


def _layer_body(adj_ref, h_ref, w_ref, out_ref):
    t = jnp.dot(adj_ref[...], h_ref[...], preferred_element_type=jnp.float32)
    t = jnp.dot(t, w_ref[...], preferred_element_type=jnp.float32)
    out_ref[...] = jnp.maximum(t, 0.0).astype(out_ref.dtype)


def _layer(adj, h, w, out_dtype, bm):
    m, k = adj.shape
    d = w.shape[1]
    return pl.pallas_call(
        _layer_body,
        grid=(pl.cdiv(m, bm),),
        in_specs=[
            pl.BlockSpec((bm, k), lambda i: (i, 0)),
            pl.BlockSpec((k, d), lambda i: (0, 0)),
            pl.BlockSpec((d, d), lambda i: (0, 0)),
        ],
        out_specs=pl.BlockSpec((bm, d), lambda i: (i, 0)),
        out_shape=jax.ShapeDtypeStruct((m, d), out_dtype),
    )(adj, h, w)


def kernel(x, adj, W0, W1, W2):
    h, adj16 = _layer1(adj, x.astype(jnp.bfloat16), W0)
    h = _layer(adj16, h, W1, jnp.bfloat16, 1024)
    return _layer(adj16, h, W2, jnp.float32, 1024)


# speedup vs baseline: 1.1712x; 1.0138x over previous
import jax
import jax.numpy as jnp
from jax.experimental import pallas as pl


_BM = 400


def _layer1_body(adj_ref, h_ref, w_ref, out_ref, adj16_ref):
    a16 = adj_ref[...].astype(jnp.bfloat16)
    adj16_ref[...] = a16
    t = jnp.dot(a16, h_ref[...], preferred_element_type=jnp.float32)
    t = jnp.dot(t, w_ref[...], preferred_element_type=jnp.float32)
    out_ref[...] = jnp.maximum(t, 0.0).astype(out_ref.dtype)


def _layer1(adj, h, w):
    m, k = adj.shape
    d = w.shape[1]
    return pl.pallas_call(
        _layer1_body,
        grid=(pl.cdiv(m, _BM),),
        in_specs=[
            pl.BlockSpec((_BM, k), lambda i: (i, 0)),
            pl.BlockSpec((k, d), lambda i: (0, 0)),
            pl.BlockSpec((d, d), lambda i: (0, 0)),
        ],
        out_specs=[
            pl.BlockSpec((_BM, d), lambda i: (i, 0)),
            pl.BlockSpec((_BM, k), lambda i: (i, 0)),
        ],
        out_shape=[
            jax.ShapeDtypeStruct((m, d), jnp.bfloat16),
            jax.ShapeDtypeStruct((m, k), jnp.bfloat16),
        ],
    )(adj, h, w)


def _layer_body(adj_ref, h_ref, w_ref, out_ref):
    t = jnp.dot(adj_ref[...], h_ref[...], preferred_element_type=jnp.float32)
    t = jnp.dot(t, w_ref[...], preferred_element_type=jnp.float32)
    out_ref[...] = jnp.maximum(t, 0.0).astype(out_ref.dtype)


def _layer(adj, h, w, out_dtype, bm):
    m, k = adj.shape
    d = w.shape[1]
    return pl.pallas_call(
        _layer_body,
        grid=(pl.cdiv(m, bm),),
        in_specs=[
            pl.BlockSpec((bm, k), lambda i: (i, 0)),
            pl.BlockSpec((k, d), lambda i: (0, 0)),
            pl.BlockSpec((d, d), lambda i: (0, 0)),
        ],
        out_specs=pl.BlockSpec((bm, d), lambda i: (i, 0)),
        out_shape=jax.ShapeDtypeStruct((m, d), out_dtype),
    )(adj, h, w)


def kernel(x, adj, W0, W1, W2):
    h, adj16 = _layer1(adj, x.astype(jnp.bfloat16), W0)
    h = _layer(adj16, h, W1, jnp.bfloat16, 1000)
    return _layer(adj16, h, W2, jnp.float32, 1000)
